# baseline (speedup 1.0000x reference)
import functools

import jax
import jax.numpy as jnp
from jax import lax
from jax.experimental import pallas as pl
from jax.experimental.pallas import tpu as pltpu
from jax.experimental.pallas import tpu_sc as plsc

N_TOTAL = 50000
E_TOTAL = 800000
T_H = 12
T_ALL = 24
HIDDEN = 64
TEMB = 16
DIFF_T = 100
MASK_RATIO = 0.1

N_PAD = 50048
NPASS = 5
N_QUARTER = 10016
ACC_ROWS = 10080
DUMMY = N_QUARTER
STRIPE = ACC_ROWS // 16
E_PAD = 802816
NCHUNK = 392
CHUNK = 128
HALF = 32
B_TC = 3128


def _sc_agg_quarter_body(h_hbm, sq_hbm, dq_hbm, zeros_hbm, agg_hbm,
                         src_buf, dst_buf, r0, r1, shared, sem0, sem1):
    c = lax.axis_index("c")
    s = lax.axis_index("s")
    row0 = s * STRIPE
    pltpu.sync_copy(sq_hbm.at[s], src_buf)
    pltpu.sync_copy(dq_hbm.at[s], dst_buf)
    pltpu.sync_copy(zeros_hbm.at[pl.ds(row0, STRIPE)], shared.at[pl.ds(row0, STRIPE)])
    off = (c * N_PAD).astype(jnp.int32)

    @pl.loop(0, NCHUNK + 1)
    def _(j):
        @pl.loop(0, CHUNK, step=16)
        def _(k):
            src_buf[j, pl.ds(k, 16)] = src_buf[j, pl.ds(k, 16)] + off

    plsc.subcore_barrier()

    pltpu.async_copy(h_hbm.at[src_buf.at[0]], r0, sem0)

    @pl.loop(0, NCHUNK, step=2)
    def _(j):
        pltpu.async_copy(h_hbm.at[src_buf.at[j + 1]], r1, sem1)
        pltpu.make_async_copy(h_hbm.at[src_buf.at[j]], r0, sem0).wait()
        pltpu.sync_copy(r0, shared.at[dst_buf.at[j]], add=True)
        pltpu.async_copy(h_hbm.at[src_buf.at[j + 2]], r0, sem0)
        pltpu.make_async_copy(h_hbm.at[src_buf.at[j + 1]], r1, sem1).wait()
        pltpu.sync_copy(r1, shared.at[dst_buf.at[j + 1]], add=True)

    pltpu.make_async_copy(h_hbm.at[src_buf.at[NCHUNK]], r0, sem0).wait()
    plsc.subcore_barrier()
    pltpu.sync_copy(shared.at[pl.ds(row0, STRIPE)],
                    agg_hbm.at[pl.ds(c * ACC_ROWS + row0, STRIPE)])


@functools.lru_cache(maxsize=None)
def _sc_kernels():
    mesh = plsc.VectorSubcoreMesh(core_axis_name="c", subcore_axis_name="s",
                                  num_cores=2, num_subcores=16)
    cp = pltpu.CompilerParams(use_tc_tiling_on_sc=False)
    agg = pl.kernel(
        _sc_agg_quarter_body,
        compiler_params=cp,
        out_type=jax.ShapeDtypeStruct((2 * ACC_ROWS, HALF), jnp.float32),
        mesh=mesh,
        scratch_types=[
            pltpu.VMEM((NCHUNK + 1, CHUNK), jnp.int32),
            pltpu.VMEM((NCHUNK, CHUNK), jnp.int32),
            pltpu.VMEM((CHUNK, HALF), jnp.float32),
            pltpu.VMEM((CHUNK, HALF), jnp.float32),
            pltpu.VMEM_SHARED((ACC_ROWS, HALF), jnp.float32),
            pltpu.SemaphoreType.DMA,
            pltpu.SemaphoreType.DMA,
        ],
    )
    return agg


def _sc_round(h_flat, src_q, dst_q, zerosA):
    k = _sc_kernels()
    return [k(h_flat, src_q[p], dst_q[p], zerosA) for p in range(NPASS)]


def _tc1_body(x_ref, noise_ref, rmask_ref, t_ref, ab_ref, fr_ref, W1_ref, b1_ref,
              h1_ref, hp_ref):
    tb = t_ref[...]
    lanes = lax.broadcasted_iota(jnp.int32, (1, 128), 1)
    ab = jnp.sum(jnp.where(tb == lanes, ab_ref[...], 0.0), axis=1, keepdims=True)
    xt = jnp.sqrt(ab) * x_ref[...] + jnp.sqrt(1.0 - ab) * noise_ref[...]
    hist_m = jnp.where(rmask_ref[...] < MASK_RATIO, 0.0, x_ref[...][:, :T_H])
    ang = tb.astype(jnp.float32) * fr_ref[...]
    h_in = jnp.concatenate(
        [xt, hist_m, jnp.zeros((B_TC, T_ALL - T_H), jnp.float32),
         jnp.sin(ang), jnp.cos(ang)], axis=1)
    h1 = jnp.maximum(
        jnp.dot(h_in, W1_ref[...], preferred_element_type=jnp.float32)
        + b1_ref[...], 0.0)
    h1_ref[...] = h1
    hp_ref[...] = jnp.stack([h1[:, :HALF], h1[:, HALF:]], axis=0)


def _gnn_layer(aggp, degp, W2, b2, h_prev):
    deg = degp[0][:, 0:1]
    inv = 1.0 / jnp.maximum(deg, 1.0)
    agg = jnp.concatenate([aggp[0], aggp[1]], axis=1) * inv
    return jnp.maximum(
        jnp.dot(agg, W2, preferred_element_type=jnp.float32) + b2, 0.0) + h_prev


def _tc2_body(aggp_ref, degp_ref, h1_ref, W2_ref, b2_ref, h2_ref, hp_ref):
    h2 = _gnn_layer(aggp_ref[...], degp_ref[...], W2_ref[...], b2_ref[...],
                    h1_ref[...])
    h2_ref[...] = h2
    hp_ref[...] = jnp.stack([h2[:, :HALF], h2[:, HALF:]], axis=0)


def _tc3_body(aggp_ref, degp_ref, h2_ref, noise_ref, W2_ref, b2_ref,
              Wout_ref, bout_ref, acc_ref):
    i = pl.program_id(0)
    h3 = _gnn_layer(aggp_ref[...], degp_ref[...], W2_ref[...], b2_ref[...],
                    h2_ref[...])
    out = jnp.dot(h3, Wout_ref[...], preferred_element_type=jnp.float32) + bout_ref[...]
    diff = out - noise_ref[...]
    rows = i * B_TC + lax.broadcasted_iota(jnp.int32, (B_TC, 1), 0)
    sq = jnp.where(rows < N_TOTAL, diff * diff, 0.0)

    @pl.when(i == 0)
    def _():
        acc_ref[...] = jnp.zeros((1, 1), jnp.float32)

    acc_ref[...] += jnp.sum(sq).reshape(1, 1)


def _row_spec(width):
    return pl.BlockSpec((B_TC, width), lambda i: (i, 0))


def _rep_spec(shape):
    return pl.BlockSpec(shape, lambda i: tuple(0 for _ in shape))


def _part_spec(width):
    return pl.BlockSpec((2, B_TC, width), lambda i: (0, i, 0))


def _stitch(blocks, width):
    out = jnp.concatenate(
        [b.reshape(2, ACC_ROWS, width)[:, :N_QUARTER] for b in blocks], axis=1)
    return out[:, :N_PAD]


def kernel(x, edge_index, batch_index, t_node, noise, rand_mask,
           W1, b1, W2, b2, Wout, bout):
    del batch_index
    f32 = jnp.float32
    grid = (N_PAD // B_TC,)

    beta = jnp.linspace(1e-4, 0.02, DIFF_T, dtype=f32)
    alphabar = jnp.cumprod(1.0 - beta)
    ab_row = jnp.zeros((1, 128), f32).at[0, :DIFF_T].set(alphabar)
    half = TEMB // 2
    freqs = jnp.exp(-jnp.log(10000.0) * jnp.arange(half, dtype=f32) / half)
    freqs = freqs.reshape(1, half)

    pad_n = N_PAD - N_TOTAL
    x2 = jnp.pad(x.reshape(N_TOTAL, T_ALL), ((0, pad_n), (0, 0)))
    noise2 = jnp.pad(noise.reshape(N_TOTAL, T_ALL), ((0, pad_n), (0, 0)))
    rmask2 = jnp.pad(rand_mask.reshape(N_TOTAL, T_H), ((0, pad_n), (0, 0)),
                     constant_values=1.0)
    t2 = jnp.pad(t_node, (0, pad_n)).reshape(N_PAD, 1)

    pad_e = E_PAD - E_TOTAL
    src = jnp.pad(edge_index[0], (0, pad_e))
    dst = jnp.pad(edge_index[1], (0, pad_e), constant_values=N_TOTAL)
    src_q, dst_q = [], []
    for p in range(NPASS):
        lo = p * N_QUARTER
        in_q = (dst >= lo) & (dst < lo + N_QUARTER)
        dst_q.append(jnp.where(in_q, dst - lo, DUMMY).reshape(16, NCHUNK, CHUNK))
        src_q.append(jnp.pad(jnp.where(in_q, src, 0).reshape(16, NCHUNK, CHUNK),
                             ((0, 0), (0, 1), (0, 0))))
    ones_tab = jnp.ones((2 * N_PAD, HALF), f32)
    src_zero = jnp.zeros((16, NCHUNK + 1, CHUNK), jnp.int32)
    zerosA = jnp.zeros((ACC_ROWS, HALF), f32)

    b1r = b1.reshape(1, HIDDEN)
    b2r = b2.reshape(1, HIDDEN)
    boutr = bout.reshape(1, T_ALL)

    h1, hp1 = pl.pallas_call(
        _tc1_body,
        grid=grid,
        in_specs=[_row_spec(T_ALL), _row_spec(T_ALL), _row_spec(T_H),
                  _row_spec(1), _rep_spec((1, 128)), _rep_spec((1, half)),
                  _rep_spec((HIDDEN, HIDDEN)), _rep_spec((1, HIDDEN))],
        out_specs=[_row_spec(HIDDEN), _part_spec(HALF)],
        out_shape=[jax.ShapeDtypeStruct((N_PAD, HIDDEN), f32),
                   jax.ShapeDtypeStruct((2, N_PAD, HALF), f32)],
    )(x2, noise2, rmask2, t2, ab_row, freqs, W1, b1r)

    degp = _stitch(_sc_round(ones_tab, [src_zero] * NPASS, dst_q, zerosA), HALF)
    agg1 = _stitch(_sc_round(hp1.reshape(2 * N_PAD, HALF), src_q, dst_q,
                             zerosA), HALF)

    h2, hp2 = pl.pallas_call(
        _tc2_body,
        grid=grid,
        in_specs=[_part_spec(HALF), _part_spec(HALF), _row_spec(HIDDEN),
                  _rep_spec((HIDDEN, HIDDEN)), _rep_spec((1, HIDDEN))],
        out_specs=[_row_spec(HIDDEN), _part_spec(HALF)],
        out_shape=[jax.ShapeDtypeStruct((N_PAD, HIDDEN), f32),
                   jax.ShapeDtypeStruct((2, N_PAD, HALF), f32)],
    )(agg1, degp, h1, W2, b2r)

    agg2 = _stitch(_sc_round(hp2.reshape(2 * N_PAD, HALF), src_q, dst_q,
                             zerosA), HALF)

    acc = pl.pallas_call(
        _tc3_body,
        grid=grid,
        in_specs=[_part_spec(HALF), _part_spec(HALF), _row_spec(HIDDEN),
                  _row_spec(T_ALL), _rep_spec((HIDDEN, HIDDEN)),
                  _rep_spec((1, HIDDEN)), _rep_spec((HIDDEN, T_ALL)),
                  _rep_spec((1, T_ALL))],
        out_specs=pl.BlockSpec((1, 1), lambda i: (0, 0)),
        out_shape=jax.ShapeDtypeStruct((1, 1), f32),
    )(agg2, degp, h2, noise2, W2, b2r, Wout, boutr)

    return acc[0, 0] / (N_TOTAL * T_ALL)
